# Initial kernel scaffold; baseline (speedup 1.0000x reference)
import jax
import jax.numpy as jnp
from jax.experimental import pallas as pl


def kernel(preds, e, shared_e, y, batch):
    raise NotImplementedError("write your pallas kernel here")


# speedup vs baseline: 31.8046x; 31.8046x over previous
import functools

import jax
import jax.numpy as jnp
from jax import lax
from jax.experimental import pallas as pl
from jax.experimental.pallas import tpu as pltpu
from jax.experimental.pallas import tpu_sc as plsc

N = 1600000
G = 4096
NCH = 4
NC = 2
NS = 16
L = 16
NW = NC * NS
CHUNK = N // NW
BLK = 10000
NB = CHUNK // BLK
VPB = BLK // L

_mesh = plsc.VectorSubcoreMesh(core_axis_name="c", subcore_axis_name="s")


def _lane_gather(x, idx):
    dnums = lax.GatherDimensionNumbers(
        offset_dims=(), collapsed_slice_dims=(0,), start_index_map=(0,))
    return lax.gather(x, idx[:, None], dnums, (1,),
                      mode=lax.GatherScatterMode.PROMISE_IN_BOUNDS)


def _cumsum16(x, iota):
    for d in (1, 2, 4, 8):
        shifted = _lane_gather(x, jnp.maximum(iota - d, 0))
        x = x + jnp.where(iota >= d, shifted, 0.0)
    return x


@functools.partial(
    pl.kernel,
    mesh=_mesh,
    out_type=jax.ShapeDtypeStruct((NW, NCH, G), jnp.float32),
    compiler_params=pltpu.CompilerParams(needs_layout_passes=False),
    scratch_types=[
        pltpu.VMEM((BLK,), jnp.int32),
        pltpu.VMEM((BLK,), jnp.float32),
        pltpu.VMEM((BLK,), jnp.float32),
        pltpu.VMEM((BLK,), jnp.float32),
        pltpu.VMEM((BLK,), jnp.float32),
        pltpu.VMEM((BLK,), jnp.float32),
        pltpu.VMEM((G,), jnp.float32),
        pltpu.VMEM((G,), jnp.float32),
        pltpu.VMEM((G,), jnp.float32),
        pltpu.VMEM((G,), jnp.float32),
    ],
)
def _sc_partials(batch_hbm, e_hbm, se_hbm, y_hbm, p0_hbm, p1_hbm, out_hbm,
                 b_buf, e_buf, se_buf, y_buf, p0_buf, p1_buf,
                 acc_a, acc_b, acc_f, acc_g):
    wid = lax.axis_index("s") * NC + lax.axis_index("c")
    base = wid * CHUNK
    zero = jnp.zeros((L,), jnp.float32)
    iota = lax.iota(jnp.int32, L)

    def zinit(i, carry):
        off = i * L
        acc_a[pl.ds(off, L)] = zero
        acc_b[pl.ds(off, L)] = zero
        acc_f[pl.ds(off, L)] = zero
        acc_g[pl.ds(off, L)] = zero
        return carry

    lax.fori_loop(0, G // L, zinit, 0)

    def do_block(bi, carry):
        blk_base = base + bi * BLK
        pltpu.sync_copy(batch_hbm.at[pl.ds(blk_base, BLK)], b_buf)
        pltpu.sync_copy(e_hbm.at[pl.ds(blk_base, BLK)], e_buf)
        pltpu.sync_copy(se_hbm.at[pl.ds(blk_base, BLK)], se_buf)
        pltpu.sync_copy(y_hbm.at[pl.ds(blk_base, BLK)], y_buf)
        pltpu.sync_copy(p0_hbm.at[pl.ds(blk_base, BLK)], p0_buf)
        pltpu.sync_copy(p1_hbm.at[pl.ds(blk_base, BLK)], p1_buf)

        def do_vreg(j, vcarry):
            off = j * L
            bv = b_buf[pl.ds(off, L)]
            ev = e_buf[pl.ds(off, L)]
            sev = se_buf[pl.ds(off, L)]
            yv = y_buf[pl.ds(off, L)]
            p0 = p0_buf[pl.ds(off, L)]
            p1 = p1_buf[pl.ds(off, L)]
            emsv = ev - sev
            d0 = p0 - yv
            d1 = p1 - (1.0 - yv)
            fv = sev * d0 * d0
            gv = emsv * d1 * d1

            bshift = _lane_gather(bv, jnp.minimum(iota + 1, L - 1))
            mb = bv != bshift
            m_end = mb | (iota == L - 1)
            ca = _cumsum16(ev, iota)
            cb = _cumsum16(emsv, iota)
            cf = _cumsum16(fv, iota)
            cg = _cumsum16(gv, iota)
            plsc.addupdate_scatter(acc_a, [bv], ca, mask=m_end)
            plsc.addupdate_scatter(acc_b, [bv], cb, mask=m_end)
            plsc.addupdate_scatter(acc_f, [bv], cf, mask=m_end)
            plsc.addupdate_scatter(acc_g, [bv], cg, mask=m_end)
            plsc.addupdate_scatter(acc_a, [bshift], -ca, mask=mb)
            plsc.addupdate_scatter(acc_b, [bshift], -cb, mask=mb)
            plsc.addupdate_scatter(acc_f, [bshift], -cf, mask=mb)
            plsc.addupdate_scatter(acc_g, [bshift], -cg, mask=mb)
            return vcarry

        return lax.fori_loop(0, VPB, do_vreg, carry)

    lax.fori_loop(0, NB, do_block, 0)

    pltpu.sync_copy(acc_a, out_hbm.at[wid, 0])
    pltpu.sync_copy(acc_b, out_hbm.at[wid, 1])
    pltpu.sync_copy(acc_f, out_hbm.at[wid, 2])
    pltpu.sync_copy(acc_g, out_hbm.at[wid, 3])


def _combine_body(x_ref, o_ref):
    t = jnp.sum(x_ref[...], axis=0)
    a = t[0:1, :]
    b = t[1:2, :]
    f = t[2:3, :]
    g = t[3:4, :]
    batches = f / (1.0 + a) + g / (1.0 + b)
    o_ref[...] = jnp.full((1, 1), jnp.sum(batches), jnp.float32)


_combine = pl.pallas_call(
    _combine_body,
    out_shape=jax.ShapeDtypeStruct((1, 1), jnp.float32),
)


def kernel(preds, e, shared_e, y, batch):
    batch = batch.astype(jnp.int32)
    p0 = preds[:, 0]
    p1 = preds[:, 1]
    partials = _sc_partials(batch, e, shared_e, y, p0, p1)
    return _combine(partials)[0, 0]
